# baseline (speedup 1.0000x reference)
import functools

import jax
import jax.numpy as jnp
from jax import lax
from jax.experimental import pallas as pl
from jax.experimental.pallas import tpu as pltpu
from jax.experimental.pallas import tpu_sc as plsc

N = 10000
NP = 10240
D = 128
K = 20
KP = 32
L = 16
NC, NS = 2, 16
NW = NC * NS
ROWS_PER_W = 313
TOTAL_ROWS = NW * ROWS_PER_W
MCHUNKS = NP // L
RCHUNKS = N // L
MV = MCHUNKS // L
U = 5
MGROUPS = MV // U
CAND = N + 2 * L
CIDCAP = MCHUNKS + L
NEG = -3.0e38
BIG = 2**30

BR = 200
BC = 2048


def _norm_body(emb_ref, out_ref):
    x = emb_ref[...]
    sq = jnp.sum(x * x, axis=1, keepdims=True)
    out_ref[...] = x * lax.rsqrt(jnp.maximum(sq, 1e-12))


def _matmul_body(a_ref, b_ref, s_ref, m_ref):
    j = pl.program_id(1)
    s = lax.dot_general(
        a_ref[...], b_ref[...],
        (((1,), (1,)), ((), ())),
        preferred_element_type=jnp.float32,
    )
    s_ref[...] = s
    m = jnp.max(s.reshape(BR, BC // L, L), axis=2)
    gchunk = j * (BC // L) + lax.broadcasted_iota(
        jnp.int32, (BR, BC // L), 1)
    m_ref[...] = jnp.where(gchunk >= RCHUNKS, NEG, m)


def _topk_sc_body(sim_hbm, mx_hbm, outv_hbm, outi_hbm,
                  rb0, rb1, rb2, rb3, mb0, mb1, mb2, mb3,
                  cd0, cd1, cv0, cv1, ci0, ci1,
                  ovals, oidx, si,
                  rs0, rs1, rs2, rs3, ms0, ms1, ms2, ms3):
    wid = lax.axis_index("s") * NC + lax.axis_index("c")
    base = wid * ROWS_PER_W
    iota16 = lax.iota(jnp.int32, L)
    neg16 = jnp.full((L,), NEG, jnp.float32)
    big16 = jnp.full((L,), BIG, jnp.int32)
    zero16 = jnp.zeros((L,), jnp.int32)
    lane0 = iota16 == 0
    rbs = (rb0, rb1, rb2, rb3)
    mbs = (mb0, mb1, mb2, mb3)
    rsems = (rs0, rs1, rs2, rs3)
    msems = (ms0, ms1, ms2, ms3)

    def valid(r):
        return jnp.logical_and(r < ROWS_PER_W, base + r < N)

    def start(r, slot):
        @pl.when(valid(r))
        def _():
            pltpu.make_async_copy(
                sim_hbm.at[base + r], rbs[slot], rsems[slot]).start()
            pltpu.make_async_copy(
                mx_hbm.at[base + r], mbs[slot], msems[slot]).start()

    def stream_pass(r, slot, half):
        rb, mb = rbs[slot], mbs[slot]
        cd = cd0 if half == 0 else cd1
        cv = cv0 if half == 0 else cv1
        ci = ci0 if half == 0 else ci1
        si[half] = jnp.int32(0)

        @pl.when(valid(r))
        def _():
            pltpu.make_async_copy(
                mx_hbm.at[base + r], mb, msems[slot]).wait()

            def p1(i, carry):
                r1, r2 = carry
                for j in range(U):
                    v = mb[pl.ds((i * U + j) * L, L)]
                    m2 = jnp.maximum(r1, v)
                    r2 = jnp.maximum(r2, jnp.minimum(r1, v))
                    r1 = m2
                return r1, r2

            r1, r2 = lax.fori_loop(0, MGROUPS, p1, (neg16, neg16))
            t = jnp.min(r2)

            def p2(i, cnt):
                c0 = i * U
                vs = [mb[pl.ds((c0 + j) * L, L)] for j in range(U)]
                ms = [v >= t for v in vs]
                ns = [plsc.all_reduce_population_count(m)[0]
                      for m in ms]
                o = cnt
                for j in range(U):
                    plsc.store_compressed(
                        cd.at[pl.ds(o, L)],
                        (c0 + j) * L + iota16, mask=ms[j])
                    o = o + ns[j]
                return o

            ccnt = lax.fori_loop(0, MGROUPS, p2, jnp.int32(0))
            cd[pl.ds(ccnt, L)] = zero16

            pltpu.make_async_copy(
                sim_hbm.at[base + r], rb, rsems[slot]).wait()
            nb = (ccnt + L - 1) // L

            def p3(b, cnt):
                ids = cd[pl.ds(b * L, L)]
                o = cnt
                for j in range(L):
                    cidj = ids[j]
                    g16 = cidj * L + iota16
                    v = plsc.load_gather(rb, [g16])
                    m = jnp.logical_and(v >= t, b * L + j < ccnt)
                    n = plsc.all_reduce_population_count(m)[0]
                    plsc.store_compressed(cv.at[pl.ds(o, L)], v,
                                          mask=m)
                    plsc.store_compressed(ci.at[pl.ds(o, L)], g16,
                                          mask=m)
                    o = o + n
                return o

            cnt = lax.fori_loop(0, nb, p3, jnp.int32(0))
            si[half] = cnt
            cv[pl.ds(cnt, L)] = neg16

    def joint_select(r0):
        cnt_a = si[0]
        cnt_b = si[1]
        nva = (cnt_a + L - 1) // L
        nvb = (cnt_b + L - 1) // L
        nvm = jnp.maximum(nva, nvb)

        def sel(k, carry):
            (av0a, av1a, ai0a, ai1a,
             av0b, av1b, ai0b, ai1b) = carry

            def scan(j, c2):
                bva, bpa, bvb, bpb = c2
                pa = j * L + iota16
                va = cv0[pl.ds(j * L, L)]
                vb = cv1[pl.ds(j * L, L)]
                beta = jnp.logical_and(va > bva, j < nva)
                betb = jnp.logical_and(vb > bvb, j < nvb)
                bva = jnp.where(beta, va, bva)
                bpa = jnp.where(beta, pa, bpa)
                bvb = jnp.where(betb, vb, bvb)
                bpb = jnp.where(betb, pa, bpb)
                return bva, bpa, bvb, bpb

            bva, bpa, bvb, bpb = lax.fori_loop(
                0, nvm, scan, (neg16, big16, neg16, big16))
            vma = jnp.max(bva)
            vmb = jnp.max(bvb)
            posa = jnp.minimum(
                jnp.min(jnp.where(bva == vma, bpa, big16)), CAND - 1)
            posb = jnp.minimum(
                jnp.min(jnp.where(bvb == vmb, bpb, big16)), CAND - 1)
            pa16 = jnp.full((L,), posa, jnp.int32)
            pb16 = jnp.full((L,), posb, jnp.int32)
            idxa = plsc.load_gather(ci0, [pa16])
            idxb = plsc.load_gather(ci1, [pb16])
            plsc.store_scatter(cv0, [pa16], neg16, mask=lane0)
            plsc.store_scatter(cv1, [pb16], neg16, mask=lane0)
            mk0 = iota16 == k
            mk1 = iota16 == k - L
            av0a = jnp.where(mk0, vma, av0a)
            av1a = jnp.where(mk1, vma, av1a)
            ai0a = jnp.where(mk0, idxa, ai0a)
            ai1a = jnp.where(mk1, idxa, ai1a)
            av0b = jnp.where(mk0, vmb, av0b)
            av1b = jnp.where(mk1, vmb, av1b)
            ai0b = jnp.where(mk0, idxb, ai0b)
            ai1b = jnp.where(mk1, idxb, ai1b)
            return (av0a, av1a, ai0a, ai1a,
                    av0b, av1b, ai0b, ai1b)

        (av0a, av1a, ai0a, ai1a,
         av0b, av1b, ai0b, ai1b) = lax.fori_loop(
            0, K, sel,
            (neg16, neg16, big16, big16,
             neg16, neg16, big16, big16))

        @pl.when(valid(r0))
        def _():
            ovals[pl.ds(r0 * KP, L)] = av0a
            ovals[pl.ds(r0 * KP + L, L)] = av1a
            oidx[pl.ds(r0 * KP, L)] = ai0a
            oidx[pl.ds(r0 * KP + L, L)] = ai1a

        @pl.when(valid(r0 + 1))
        def _():
            ovals[pl.ds((r0 + 1) * KP, L)] = av0b
            ovals[pl.ds((r0 + 1) * KP + L, L)] = av1b
            oidx[pl.ds((r0 + 1) * KP, L)] = ai0b
            oidx[pl.ds((r0 + 1) * KP + L, L)] = ai1b

    for s in range(4):
        start(s, s)

    def outer(i, _):
        r0 = i * 4
        stream_pass(r0, 0, 0)
        stream_pass(r0 + 1, 1, 1)
        start(r0 + 4, 0)
        start(r0 + 5, 1)
        joint_select(r0)
        stream_pass(r0 + 2, 2, 0)
        stream_pass(r0 + 3, 3, 1)
        start(r0 + 6, 2)
        start(r0 + 7, 3)
        joint_select(r0 + 2)
        return 0

    lax.fori_loop(0, (ROWS_PER_W + 3) // 4, outer, 0)
    pltpu.sync_copy(ovals, outv_hbm.at[pl.ds(base * KP, ROWS_PER_W * KP)])
    pltpu.sync_copy(oidx, outi_hbm.at[pl.ds(base * KP, ROWS_PER_W * KP)])


def _build_topk_sc():
    return functools.partial(
        pl.kernel,
        out_type=[
            jax.ShapeDtypeStruct((TOTAL_ROWS * KP,), jnp.float32),
            jax.ShapeDtypeStruct((TOTAL_ROWS * KP,), jnp.int32),
        ],
        mesh=plsc.VectorSubcoreMesh(core_axis_name="c", subcore_axis_name="s",
                                    num_cores=NC, num_subcores=NS),
        compiler_params=pltpu.CompilerParams(needs_layout_passes=False),
        scratch_types=[
            pltpu.VMEM((NP,), jnp.float32),
            pltpu.VMEM((NP,), jnp.float32),
            pltpu.VMEM((NP,), jnp.float32),
            pltpu.VMEM((NP,), jnp.float32),
            pltpu.VMEM((MCHUNKS,), jnp.float32),
            pltpu.VMEM((MCHUNKS,), jnp.float32),
            pltpu.VMEM((MCHUNKS,), jnp.float32),
            pltpu.VMEM((MCHUNKS,), jnp.float32),
            pltpu.VMEM((CIDCAP,), jnp.int32),
            pltpu.VMEM((CIDCAP,), jnp.int32),
            pltpu.VMEM((CAND,), jnp.float32),
            pltpu.VMEM((CAND,), jnp.float32),
            pltpu.VMEM((CAND,), jnp.int32),
            pltpu.VMEM((CAND,), jnp.int32),
            pltpu.VMEM((ROWS_PER_W * KP,), jnp.float32),
            pltpu.VMEM((ROWS_PER_W * KP,), jnp.int32),
            pltpu.SMEM((2,), jnp.int32),
            pltpu.SemaphoreType.DMA,
            pltpu.SemaphoreType.DMA,
            pltpu.SemaphoreType.DMA,
            pltpu.SemaphoreType.DMA,
            pltpu.SemaphoreType.DMA,
            pltpu.SemaphoreType.DMA,
            pltpu.SemaphoreType.DMA,
            pltpu.SemaphoreType.DMA,
        ],
    )(_topk_sc_body)


def kernel(embeddings):
    emb_pad = jnp.pad(embeddings, ((0, NP - N), (0, 0)))
    norm = pl.pallas_call(
        _norm_body,
        out_shape=jax.ShapeDtypeStruct((NP, D), jnp.float32),
    )(emb_pad)

    sim, mx = pl.pallas_call(
        _matmul_body,
        grid=(N // BR, NP // BC),
        in_specs=[
            pl.BlockSpec((BR, D), lambda i, j: (i, 0)),
            pl.BlockSpec((BC, D), lambda i, j: (j, 0)),
        ],
        out_specs=[
            pl.BlockSpec((BR, BC), lambda i, j: (i, j)),
            pl.BlockSpec((BR, BC // L), lambda i, j: (i, j)),
        ],
        out_shape=[
            jax.ShapeDtypeStruct((N, NP), jnp.float32),
            jax.ShapeDtypeStruct((N, MCHUNKS), jnp.float32),
        ],
    )(norm, norm)

    vflat, iflat = _build_topk_sc()(sim, mx)
    vals = vflat.reshape(TOTAL_ROWS, KP)[:N, :K]
    idx = iflat.reshape(TOTAL_ROWS, KP)[:N, :K]
    return vals, idx


# speedup vs baseline: 2.4678x; 2.4678x over previous
import functools

import jax
import jax.numpy as jnp
from jax import lax
from jax.experimental import pallas as pl
from jax.experimental.pallas import tpu as pltpu
from jax.experimental.pallas import tpu_sc as plsc

N = 10000
NP = 10240
D = 128
K = 20
KP = 32
L = 16
NC, NS = 2, 16
NW = NC * NS
ROWS_PER_W = 313
TOTAL_ROWS = NW * ROWS_PER_W
MCHUNKS = NP // L
RCHUNKS = N // L
MV = MCHUNKS // L
U = 5
MGROUPS = MV // U
CAND = N + 2 * L
CIDCAP = MCHUNKS + L
NEG = -3.0e38
BIG = 2**30

BR = 400
BC = 2048


def _norm_body(emb_ref, out_ref):
    x = emb_ref[...]
    sq = jnp.sum(x * x, axis=1, keepdims=True)
    out_ref[...] = x * lax.rsqrt(jnp.maximum(sq, 1e-12))


def _matmul_body(a_ref, b_ref, s_ref, m_ref):
    j = pl.program_id(1)
    a = a_ref[...]
    b = b_ref[...]
    s_ref[...] = lax.dot_general(
        a, b, (((1,), (1,)), ((), ())),
        preferred_element_type=jnp.float32,
    )
    b3 = b.reshape(BC // L, L, D)
    m = lax.dot_general(
        a, b3[:, 0, :], (((1,), (1,)), ((), ())),
        preferred_element_type=jnp.float32)
    for l in range(1, L):
        m = jnp.maximum(m, lax.dot_general(
            a, b3[:, l, :], (((1,), (1,)), ((), ())),
            preferred_element_type=jnp.float32))
    gchunk = j * (BC // L) + lax.broadcasted_iota(
        jnp.int32, (BR, BC // L), 1)
    m_ref[...] = jnp.where(gchunk >= RCHUNKS, NEG, m)


def _topk_sc_body(sim_hbm, mx_hbm, outv_hbm, outi_hbm,
                  rb0, rb1, rb2, rb3, mb0, mb1, mb2, mb3,
                  cd0, cd1, cv0, cv1, ci0, ci1,
                  ovals, oidx, si,
                  rs0, rs1, rs2, rs3, ms0, ms1, ms2, ms3):
    wid = lax.axis_index("s") * NC + lax.axis_index("c")
    base = wid * ROWS_PER_W
    iota16 = lax.iota(jnp.int32, L)
    neg16 = jnp.full((L,), NEG, jnp.float32)
    big16 = jnp.full((L,), BIG, jnp.int32)
    zero16 = jnp.zeros((L,), jnp.int32)
    lane0 = iota16 == 0
    rbs = (rb0, rb1, rb2, rb3)
    mbs = (mb0, mb1, mb2, mb3)
    rsems = (rs0, rs1, rs2, rs3)
    msems = (ms0, ms1, ms2, ms3)

    def valid(r):
        return jnp.logical_and(r < ROWS_PER_W, base + r < N)

    def start(r, slot):
        @pl.when(valid(r))
        def _():
            pltpu.make_async_copy(
                sim_hbm.at[base + r], rbs[slot], rsems[slot]).start()
            pltpu.make_async_copy(
                mx_hbm.at[base + r], mbs[slot], msems[slot]).start()

    def stream_pass(r, slot, half):
        rb, mb = rbs[slot], mbs[slot]
        cd = cd0 if half == 0 else cd1
        cv = cv0 if half == 0 else cv1
        ci = ci0 if half == 0 else ci1
        si[half] = jnp.int32(0)

        @pl.when(valid(r))
        def _():
            pltpu.make_async_copy(
                mx_hbm.at[base + r], mb, msems[slot]).wait()

            def p1(i, carry):
                r1, r2 = carry
                for j in range(U):
                    v = mb[pl.ds((i * U + j) * L, L)]
                    m2 = jnp.maximum(r1, v)
                    r2 = jnp.maximum(r2, jnp.minimum(r1, v))
                    r1 = m2
                return r1, r2

            r1, r2 = lax.fori_loop(0, MGROUPS, p1, (neg16, neg16))
            t = jnp.min(r2)
            t = t - jnp.abs(t) * 1e-5 - 1e-30

            def p2(i, cnt):
                c0 = i * U
                vs = [mb[pl.ds((c0 + j) * L, L)] for j in range(U)]
                ms = [v >= t for v in vs]
                ns = [plsc.all_reduce_population_count(m)[0]
                      for m in ms]
                o = cnt
                for j in range(U):
                    plsc.store_compressed(
                        cd.at[pl.ds(o, L)],
                        (c0 + j) * L + iota16, mask=ms[j])
                    o = o + ns[j]
                return o

            ccnt = lax.fori_loop(0, MGROUPS, p2, jnp.int32(0))
            cd[pl.ds(ccnt, L)] = zero16

            pltpu.make_async_copy(
                sim_hbm.at[base + r], rb, rsems[slot]).wait()
            nb = (ccnt + L - 1) // L

            def p3(b, cnt):
                ids = cd[pl.ds(b * L, L)]
                o = cnt
                for j in range(L):
                    cidj = ids[j]
                    g16 = cidj * L + iota16
                    v = plsc.load_gather(rb, [g16])
                    m = jnp.logical_and(v >= t, b * L + j < ccnt)
                    n = plsc.all_reduce_population_count(m)[0]
                    plsc.store_compressed(cv.at[pl.ds(o, L)], v,
                                          mask=m)
                    plsc.store_compressed(ci.at[pl.ds(o, L)], g16,
                                          mask=m)
                    o = o + n
                return o

            cnt = lax.fori_loop(0, nb, p3, jnp.int32(0))
            si[half] = cnt
            cv[pl.ds(cnt, L)] = neg16

    def joint_select(r0):
        cnt_a = si[0]
        cnt_b = si[1]
        nva = (cnt_a + L - 1) // L
        nvb = (cnt_b + L - 1) // L
        nvm = jnp.maximum(nva, nvb)

        def sel(k, carry):
            (av0a, av1a, ai0a, ai1a,
             av0b, av1b, ai0b, ai1b) = carry

            def scan(j, c2):
                bva, bpa, bvb, bpb = c2
                pa = j * L + iota16
                va = cv0[pl.ds(j * L, L)]
                vb = cv1[pl.ds(j * L, L)]
                beta = jnp.logical_and(va > bva, j < nva)
                betb = jnp.logical_and(vb > bvb, j < nvb)
                bva = jnp.where(beta, va, bva)
                bpa = jnp.where(beta, pa, bpa)
                bvb = jnp.where(betb, vb, bvb)
                bpb = jnp.where(betb, pa, bpb)
                return bva, bpa, bvb, bpb

            bva, bpa, bvb, bpb = lax.fori_loop(
                0, nvm, scan, (neg16, big16, neg16, big16))
            vma = jnp.max(bva)
            vmb = jnp.max(bvb)
            posa = jnp.minimum(
                jnp.min(jnp.where(bva == vma, bpa, big16)), CAND - 1)
            posb = jnp.minimum(
                jnp.min(jnp.where(bvb == vmb, bpb, big16)), CAND - 1)
            pa16 = jnp.full((L,), posa, jnp.int32)
            pb16 = jnp.full((L,), posb, jnp.int32)
            idxa = plsc.load_gather(ci0, [pa16])
            idxb = plsc.load_gather(ci1, [pb16])
            plsc.store_scatter(cv0, [pa16], neg16, mask=lane0)
            plsc.store_scatter(cv1, [pb16], neg16, mask=lane0)
            mk0 = iota16 == k
            mk1 = iota16 == k - L
            av0a = jnp.where(mk0, vma, av0a)
            av1a = jnp.where(mk1, vma, av1a)
            ai0a = jnp.where(mk0, idxa, ai0a)
            ai1a = jnp.where(mk1, idxa, ai1a)
            av0b = jnp.where(mk0, vmb, av0b)
            av1b = jnp.where(mk1, vmb, av1b)
            ai0b = jnp.where(mk0, idxb, ai0b)
            ai1b = jnp.where(mk1, idxb, ai1b)
            return (av0a, av1a, ai0a, ai1a,
                    av0b, av1b, ai0b, ai1b)

        (av0a, av1a, ai0a, ai1a,
         av0b, av1b, ai0b, ai1b) = lax.fori_loop(
            0, K, sel,
            (neg16, neg16, big16, big16,
             neg16, neg16, big16, big16))

        @pl.when(valid(r0))
        def _():
            ovals[pl.ds(r0 * KP, L)] = av0a
            ovals[pl.ds(r0 * KP + L, L)] = av1a
            oidx[pl.ds(r0 * KP, L)] = ai0a
            oidx[pl.ds(r0 * KP + L, L)] = ai1a

        @pl.when(valid(r0 + 1))
        def _():
            ovals[pl.ds((r0 + 1) * KP, L)] = av0b
            ovals[pl.ds((r0 + 1) * KP + L, L)] = av1b
            oidx[pl.ds((r0 + 1) * KP, L)] = ai0b
            oidx[pl.ds((r0 + 1) * KP + L, L)] = ai1b

    for s in range(4):
        start(s, s)

    def outer(i, _):
        r0 = i * 4
        stream_pass(r0, 0, 0)
        stream_pass(r0 + 1, 1, 1)
        start(r0 + 4, 0)
        start(r0 + 5, 1)
        joint_select(r0)
        stream_pass(r0 + 2, 2, 0)
        stream_pass(r0 + 3, 3, 1)
        start(r0 + 6, 2)
        start(r0 + 7, 3)
        joint_select(r0 + 2)
        return 0

    lax.fori_loop(0, (ROWS_PER_W + 3) // 4, outer, 0)
    pltpu.sync_copy(ovals, outv_hbm.at[pl.ds(base * KP, ROWS_PER_W * KP)])
    pltpu.sync_copy(oidx, outi_hbm.at[pl.ds(base * KP, ROWS_PER_W * KP)])


def _build_topk_sc():
    return functools.partial(
        pl.kernel,
        out_type=[
            jax.ShapeDtypeStruct((TOTAL_ROWS * KP,), jnp.float32),
            jax.ShapeDtypeStruct((TOTAL_ROWS * KP,), jnp.int32),
        ],
        mesh=plsc.VectorSubcoreMesh(core_axis_name="c", subcore_axis_name="s",
                                    num_cores=NC, num_subcores=NS),
        compiler_params=pltpu.CompilerParams(needs_layout_passes=False),
        scratch_types=[
            pltpu.VMEM((NP,), jnp.float32),
            pltpu.VMEM((NP,), jnp.float32),
            pltpu.VMEM((NP,), jnp.float32),
            pltpu.VMEM((NP,), jnp.float32),
            pltpu.VMEM((MCHUNKS,), jnp.float32),
            pltpu.VMEM((MCHUNKS,), jnp.float32),
            pltpu.VMEM((MCHUNKS,), jnp.float32),
            pltpu.VMEM((MCHUNKS,), jnp.float32),
            pltpu.VMEM((CIDCAP,), jnp.int32),
            pltpu.VMEM((CIDCAP,), jnp.int32),
            pltpu.VMEM((CAND,), jnp.float32),
            pltpu.VMEM((CAND,), jnp.float32),
            pltpu.VMEM((CAND,), jnp.int32),
            pltpu.VMEM((CAND,), jnp.int32),
            pltpu.VMEM((ROWS_PER_W * KP,), jnp.float32),
            pltpu.VMEM((ROWS_PER_W * KP,), jnp.int32),
            pltpu.SMEM((2,), jnp.int32),
            pltpu.SemaphoreType.DMA,
            pltpu.SemaphoreType.DMA,
            pltpu.SemaphoreType.DMA,
            pltpu.SemaphoreType.DMA,
            pltpu.SemaphoreType.DMA,
            pltpu.SemaphoreType.DMA,
            pltpu.SemaphoreType.DMA,
            pltpu.SemaphoreType.DMA,
        ],
    )(_topk_sc_body)


def kernel(embeddings):
    emb_pad = jnp.pad(embeddings, ((0, NP - N), (0, 0)))
    norm = pl.pallas_call(
        _norm_body,
        out_shape=jax.ShapeDtypeStruct((NP, D), jnp.float32),
    )(emb_pad)

    sim, mx = pl.pallas_call(
        _matmul_body,
        grid=(N // BR, NP // BC),
        in_specs=[
            pl.BlockSpec((BR, D), lambda i, j: (i, 0)),
            pl.BlockSpec((BC, D), lambda i, j: (j, 0)),
        ],
        out_specs=[
            pl.BlockSpec((BR, BC), lambda i, j: (i, j)),
            pl.BlockSpec((BR, BC // L), lambda i, j: (i, j)),
        ],
        out_shape=[
            jax.ShapeDtypeStruct((N, NP), jnp.float32),
            jax.ShapeDtypeStruct((N, MCHUNKS), jnp.float32),
        ],
    )(norm, norm)

    vflat, iflat = _build_topk_sc()(sim, mx)
    vals = vflat.reshape(TOTAL_ROWS, KP)[:N, :K]
    idx = iflat.reshape(TOTAL_ROWS, KP)[:N, :K]
    return vals, idx


# speedup vs baseline: 2.5622x; 1.0383x over previous
import functools

import jax
import jax.numpy as jnp
from jax import lax
from jax.experimental import pallas as pl
from jax.experimental.pallas import tpu as pltpu
from jax.experimental.pallas import tpu_sc as plsc

N = 10000
NP = 10240
D = 128
K = 20
KP = 32
L = 16
NC, NS = 2, 16
NW = NC * NS
NH = N // 2
ROWS_PER_W = 157
TOTAL_ROWS = NW * ROWS_PER_W
MCHUNKS = NP // L
RCHUNKS = N // L
MV = MCHUNKS // L
U = 5
MGROUPS = MV // U
CAND = N + 2 * L
CIDCAP = MCHUNKS + L
NEG = -3.0e38
BIG = 2**30

BR = 200
BC = 2048


def _norm_body(emb_ref, out_ref):
    x = emb_ref[...]
    sq = jnp.sum(x * x, axis=1, keepdims=True)
    out_ref[...] = x * lax.rsqrt(jnp.maximum(sq, 1e-12))


def _matmul_body(a_ref, b_ref, s_ref, m_ref):
    j = pl.program_id(1)
    a = a_ref[...]
    b = b_ref[...]
    s_ref[...] = lax.dot_general(
        a, b, (((1,), (1,)), ((), ())),
        preferred_element_type=jnp.float32,
    )
    b3 = b.reshape(BC // L, L, D)
    m = lax.dot_general(
        a, b3[:, 0, :], (((1,), (1,)), ((), ())),
        preferred_element_type=jnp.float32)
    for l in range(1, L):
        m = jnp.maximum(m, lax.dot_general(
            a, b3[:, l, :], (((1,), (1,)), ((), ())),
            preferred_element_type=jnp.float32))
    gchunk = j * (BC // L) + lax.broadcasted_iota(
        jnp.int32, (BR, BC // L), 1)
    m_ref[...] = jnp.where(gchunk >= RCHUNKS, NEG, m)


def _topk_sc_body(sim_hbm, mx_hbm, outv_hbm, outi_hbm,
                  rb0, rb1, rb2, rb3, mb0, mb1, mb2, mb3,
                  cd0, cd1, cv0, cv1, ci0, ci1,
                  ovals, oidx, si,
                  rs0, rs1, rs2, rs3, ms0, ms1, ms2, ms3):
    wid = lax.axis_index("s") * NC + lax.axis_index("c")
    base = wid * ROWS_PER_W
    iota16 = lax.iota(jnp.int32, L)
    neg16 = jnp.full((L,), NEG, jnp.float32)
    big16 = jnp.full((L,), BIG, jnp.int32)
    zero16 = jnp.zeros((L,), jnp.int32)
    lane0 = iota16 == 0
    rbs = (rb0, rb1, rb2, rb3)
    mbs = (mb0, mb1, mb2, mb3)
    rsems = (rs0, rs1, rs2, rs3)
    msems = (ms0, ms1, ms2, ms3)

    def valid(r):
        return jnp.logical_and(r < ROWS_PER_W, base + r < NH)

    def start(r, slot):
        @pl.when(valid(r))
        def _():
            pltpu.make_async_copy(
                sim_hbm.at[base + r], rbs[slot], rsems[slot]).start()
            pltpu.make_async_copy(
                mx_hbm.at[base + r], mbs[slot], msems[slot]).start()

    def stream_pass(r, slot, half):
        rb, mb = rbs[slot], mbs[slot]
        cd = cd0 if half == 0 else cd1
        cv = cv0 if half == 0 else cv1
        ci = ci0 if half == 0 else ci1
        si[half] = jnp.int32(0)

        @pl.when(valid(r))
        def _():
            pltpu.make_async_copy(
                mx_hbm.at[base + r], mb, msems[slot]).wait()

            def p1(i, carry):
                r1, r2 = carry
                for j in range(U):
                    v = mb[pl.ds((i * U + j) * L, L)]
                    m2 = jnp.maximum(r1, v)
                    r2 = jnp.maximum(r2, jnp.minimum(r1, v))
                    r1 = m2
                return r1, r2

            r1, r2 = lax.fori_loop(0, MGROUPS, p1, (neg16, neg16))
            t = jnp.min(r2)
            t = t - jnp.abs(t) * 1e-5 - 1e-30

            def p2(i, cnt):
                c0 = i * U
                vs = [mb[pl.ds((c0 + j) * L, L)] for j in range(U)]
                ms = [v >= t for v in vs]
                ns = [plsc.all_reduce_population_count(m)[0]
                      for m in ms]
                o = cnt
                for j in range(U):
                    plsc.store_compressed(
                        cd.at[pl.ds(o, L)],
                        (c0 + j) * L + iota16, mask=ms[j])
                    o = o + ns[j]
                return o

            ccnt = lax.fori_loop(0, MGROUPS, p2, jnp.int32(0))
            cd[pl.ds(ccnt, L)] = zero16

            pltpu.make_async_copy(
                sim_hbm.at[base + r], rb, rsems[slot]).wait()
            nb = (ccnt + L - 1) // L

            def p3(b, cnt):
                ids = cd[pl.ds(b * L, L)]
                o = cnt
                for j in range(L):
                    cidj = ids[j]
                    g16 = cidj * L + iota16
                    v = plsc.load_gather(rb, [g16])
                    m = jnp.logical_and(v >= t, b * L + j < ccnt)
                    n = plsc.all_reduce_population_count(m)[0]
                    plsc.store_compressed(cv.at[pl.ds(o, L)], v,
                                          mask=m)
                    plsc.store_compressed(ci.at[pl.ds(o, L)], g16,
                                          mask=m)
                    o = o + n
                return o

            cnt = lax.fori_loop(0, nb, p3, jnp.int32(0))
            si[half] = cnt
            cv[pl.ds(cnt, L)] = neg16

    def joint_select(r0):
        cnt_a = si[0]
        cnt_b = si[1]
        nva = (cnt_a + L - 1) // L
        nvb = (cnt_b + L - 1) // L
        nvm = jnp.maximum(nva, nvb)

        def sel(k, carry):
            (av0a, av1a, ai0a, ai1a,
             av0b, av1b, ai0b, ai1b) = carry

            def scan(j, c2):
                bva, bpa, bvb, bpb = c2
                pa = j * L + iota16
                va = cv0[pl.ds(j * L, L)]
                vb = cv1[pl.ds(j * L, L)]
                beta = jnp.logical_and(va > bva, j < nva)
                betb = jnp.logical_and(vb > bvb, j < nvb)
                bva = jnp.where(beta, va, bva)
                bpa = jnp.where(beta, pa, bpa)
                bvb = jnp.where(betb, vb, bvb)
                bpb = jnp.where(betb, pa, bpb)
                return bva, bpa, bvb, bpb

            bva, bpa, bvb, bpb = lax.fori_loop(
                0, nvm, scan, (neg16, big16, neg16, big16))
            vma = jnp.max(bva)
            vmb = jnp.max(bvb)
            posa = jnp.minimum(
                jnp.min(jnp.where(bva == vma, bpa, big16)), CAND - 1)
            posb = jnp.minimum(
                jnp.min(jnp.where(bvb == vmb, bpb, big16)), CAND - 1)
            pa16 = jnp.full((L,), posa, jnp.int32)
            pb16 = jnp.full((L,), posb, jnp.int32)
            idxa = plsc.load_gather(ci0, [pa16])
            idxb = plsc.load_gather(ci1, [pb16])
            plsc.store_scatter(cv0, [pa16], neg16, mask=lane0)
            plsc.store_scatter(cv1, [pb16], neg16, mask=lane0)
            mk0 = iota16 == k
            mk1 = iota16 == k - L
            av0a = jnp.where(mk0, vma, av0a)
            av1a = jnp.where(mk1, vma, av1a)
            ai0a = jnp.where(mk0, idxa, ai0a)
            ai1a = jnp.where(mk1, idxa, ai1a)
            av0b = jnp.where(mk0, vmb, av0b)
            av1b = jnp.where(mk1, vmb, av1b)
            ai0b = jnp.where(mk0, idxb, ai0b)
            ai1b = jnp.where(mk1, idxb, ai1b)
            return (av0a, av1a, ai0a, ai1a,
                    av0b, av1b, ai0b, ai1b)

        (av0a, av1a, ai0a, ai1a,
         av0b, av1b, ai0b, ai1b) = lax.fori_loop(
            0, K, sel,
            (neg16, neg16, big16, big16,
             neg16, neg16, big16, big16))

        @pl.when(valid(r0))
        def _():
            ovals[pl.ds(r0 * KP, L)] = av0a
            ovals[pl.ds(r0 * KP + L, L)] = av1a
            oidx[pl.ds(r0 * KP, L)] = ai0a
            oidx[pl.ds(r0 * KP + L, L)] = ai1a

        @pl.when(valid(r0 + 1))
        def _():
            ovals[pl.ds((r0 + 1) * KP, L)] = av0b
            ovals[pl.ds((r0 + 1) * KP + L, L)] = av1b
            oidx[pl.ds((r0 + 1) * KP, L)] = ai0b
            oidx[pl.ds((r0 + 1) * KP + L, L)] = ai1b

    for s in range(4):
        start(s, s)

    def outer(i, _):
        r0 = i * 4
        stream_pass(r0, 0, 0)
        stream_pass(r0 + 1, 1, 1)
        start(r0 + 4, 0)
        start(r0 + 5, 1)
        joint_select(r0)
        stream_pass(r0 + 2, 2, 0)
        stream_pass(r0 + 3, 3, 1)
        start(r0 + 6, 2)
        start(r0 + 7, 3)
        joint_select(r0 + 2)
        return 0

    lax.fori_loop(0, (ROWS_PER_W + 3) // 4, outer, 0)
    pltpu.sync_copy(ovals, outv_hbm.at[pl.ds(base * KP, ROWS_PER_W * KP)])
    pltpu.sync_copy(oidx, outi_hbm.at[pl.ds(base * KP, ROWS_PER_W * KP)])


def _build_topk_sc():
    return functools.partial(
        pl.kernel,
        out_type=[
            jax.ShapeDtypeStruct((TOTAL_ROWS * KP,), jnp.float32),
            jax.ShapeDtypeStruct((TOTAL_ROWS * KP,), jnp.int32),
        ],
        mesh=plsc.VectorSubcoreMesh(core_axis_name="c", subcore_axis_name="s",
                                    num_cores=NC, num_subcores=NS),
        compiler_params=pltpu.CompilerParams(needs_layout_passes=False),
        scratch_types=[
            pltpu.VMEM((NP,), jnp.float32),
            pltpu.VMEM((NP,), jnp.float32),
            pltpu.VMEM((NP,), jnp.float32),
            pltpu.VMEM((NP,), jnp.float32),
            pltpu.VMEM((MCHUNKS,), jnp.float32),
            pltpu.VMEM((MCHUNKS,), jnp.float32),
            pltpu.VMEM((MCHUNKS,), jnp.float32),
            pltpu.VMEM((MCHUNKS,), jnp.float32),
            pltpu.VMEM((CIDCAP,), jnp.int32),
            pltpu.VMEM((CIDCAP,), jnp.int32),
            pltpu.VMEM((CAND,), jnp.float32),
            pltpu.VMEM((CAND,), jnp.float32),
            pltpu.VMEM((CAND,), jnp.int32),
            pltpu.VMEM((CAND,), jnp.int32),
            pltpu.VMEM((ROWS_PER_W * KP,), jnp.float32),
            pltpu.VMEM((ROWS_PER_W * KP,), jnp.int32),
            pltpu.SMEM((2,), jnp.int32),
            pltpu.SemaphoreType.DMA,
            pltpu.SemaphoreType.DMA,
            pltpu.SemaphoreType.DMA,
            pltpu.SemaphoreType.DMA,
            pltpu.SemaphoreType.DMA,
            pltpu.SemaphoreType.DMA,
            pltpu.SemaphoreType.DMA,
            pltpu.SemaphoreType.DMA,
        ],
    )(_topk_sc_body)


def _matmul_half(norm, a_half):
    return pl.pallas_call(
        _matmul_body,
        grid=(NH // BR, NP // BC),
        in_specs=[
            pl.BlockSpec((BR, D), lambda i, j: (i, 0)),
            pl.BlockSpec((BC, D), lambda i, j: (j, 0)),
        ],
        out_specs=[
            pl.BlockSpec((BR, BC), lambda i, j: (i, j)),
            pl.BlockSpec((BR, BC // L), lambda i, j: (i, j)),
        ],
        out_shape=[
            jax.ShapeDtypeStruct((NH, NP), jnp.float32),
            jax.ShapeDtypeStruct((NH, MCHUNKS), jnp.float32),
        ],
    )(a_half, norm)


def kernel(embeddings):
    emb_pad = jnp.pad(embeddings, ((0, NP - N), (0, 0)))
    norm = pl.pallas_call(
        _norm_body,
        out_shape=jax.ShapeDtypeStruct((NP, D), jnp.float32),
    )(emb_pad)

    topk = _build_topk_sc()
    sim_a, mx_a = _matmul_half(norm, norm[:NH])
    va, ia = topk(sim_a, mx_a)
    sim_b, mx_b = _matmul_half(norm, norm[NH:2 * NH])
    vb, ib = topk(sim_b, mx_b)
    vals = jnp.concatenate([
        va.reshape(TOTAL_ROWS, KP)[:NH, :K],
        vb.reshape(TOTAL_ROWS, KP)[:NH, :K]])
    idx = jnp.concatenate([
        ia.reshape(TOTAL_ROWS, KP)[:NH, :K],
        ib.reshape(TOTAL_ROWS, KP)[:NH, :K]])
    return vals, idx


# speedup vs baseline: 2.8568x; 1.1149x over previous
import functools

import jax
import jax.numpy as jnp
from jax import lax
from jax.experimental import pallas as pl
from jax.experimental.pallas import tpu as pltpu
from jax.experimental.pallas import tpu_sc as plsc

N = 10000
NP = 10240
D = 128
K = 20
KP = 32
L = 16
NC, NS = 2, 16
NW = NC * NS
NH = N // 5
ROWS_PER_W = 63
TOTAL_ROWS = NW * ROWS_PER_W
MCHUNKS = NP // L
RCHUNKS = N // L
MV = MCHUNKS // L
U = 5
MGROUPS = MV // U
CAND = N + 2 * L
CIDCAP = MCHUNKS + L
NEG = -3.0e38
BIG = 2**30

BR = 400
BC = 2048


def _norm_body(emb_ref, out_ref):
    x = emb_ref[...]
    sq = jnp.sum(x * x, axis=1, keepdims=True)
    out_ref[...] = x * lax.rsqrt(jnp.maximum(sq, 1e-12))


def _matmul_body(a_ref, b_ref, s_ref, m_ref):
    j = pl.program_id(1)
    a = a_ref[...]
    b = b_ref[...]
    s_ref[...] = lax.dot_general(
        a, b, (((1,), (1,)), ((), ())),
        preferred_element_type=jnp.float32,
    )
    b3 = b.reshape(BC // L, L, D)
    m = lax.dot_general(
        a, b3[:, 0, :], (((1,), (1,)), ((), ())),
        preferred_element_type=jnp.float32)
    for l in range(1, L):
        m = jnp.maximum(m, lax.dot_general(
            a, b3[:, l, :], (((1,), (1,)), ((), ())),
            preferred_element_type=jnp.float32))
    gchunk = j * (BC // L) + lax.broadcasted_iota(
        jnp.int32, (BR, BC // L), 1)
    m_ref[...] = jnp.where(gchunk >= RCHUNKS, NEG, m)


def _topk_sc_body(sim_hbm, mx_hbm, outv_hbm, outi_hbm,
                  rb0, rb1, rb2, rb3, mb0, mb1, mb2, mb3,
                  cd0, cd1, cv0, cv1, ci0, ci1,
                  ovals, oidx, si,
                  rs0, rs1, rs2, rs3, ms0, ms1, ms2, ms3):
    wid = lax.axis_index("s") * NC + lax.axis_index("c")
    base = wid * ROWS_PER_W
    iota16 = lax.iota(jnp.int32, L)
    neg16 = jnp.full((L,), NEG, jnp.float32)
    big16 = jnp.full((L,), BIG, jnp.int32)
    zero16 = jnp.zeros((L,), jnp.int32)
    lane0 = iota16 == 0
    rbs = (rb0, rb1, rb2, rb3)
    mbs = (mb0, mb1, mb2, mb3)
    rsems = (rs0, rs1, rs2, rs3)
    msems = (ms0, ms1, ms2, ms3)

    def valid(r):
        return jnp.logical_and(r < ROWS_PER_W, base + r < NH)

    def start(r, slot):
        @pl.when(valid(r))
        def _():
            pltpu.make_async_copy(
                sim_hbm.at[base + r], rbs[slot], rsems[slot]).start()
            pltpu.make_async_copy(
                mx_hbm.at[base + r], mbs[slot], msems[slot]).start()

    def stream_pass(r, slot, half):
        rb, mb = rbs[slot], mbs[slot]
        cd = cd0 if half == 0 else cd1
        cv = cv0 if half == 0 else cv1
        ci = ci0 if half == 0 else ci1
        si[half] = jnp.int32(0)

        @pl.when(valid(r))
        def _():
            pltpu.make_async_copy(
                mx_hbm.at[base + r], mb, msems[slot]).wait()

            def p1(i, carry):
                r1, r2 = carry
                for j in range(U):
                    v = mb[pl.ds((i * U + j) * L, L)]
                    m2 = jnp.maximum(r1, v)
                    r2 = jnp.maximum(r2, jnp.minimum(r1, v))
                    r1 = m2
                return r1, r2

            r1, r2 = lax.fori_loop(0, MGROUPS, p1, (neg16, neg16))
            t = jnp.min(r2)
            t = t - jnp.abs(t) * 1e-5 - 1e-30

            def p2(i, cnt):
                c0 = i * U
                vs = [mb[pl.ds((c0 + j) * L, L)] for j in range(U)]
                ms = [v >= t for v in vs]
                ns = [plsc.all_reduce_population_count(m)[0]
                      for m in ms]
                o = cnt
                for j in range(U):
                    plsc.store_compressed(
                        cd.at[pl.ds(o, L)],
                        (c0 + j) * L + iota16, mask=ms[j])
                    o = o + ns[j]
                return o

            ccnt = lax.fori_loop(0, MGROUPS, p2, jnp.int32(0))
            cd[pl.ds(ccnt, L)] = zero16

            pltpu.make_async_copy(
                sim_hbm.at[base + r], rb, rsems[slot]).wait()
            nb = (ccnt + L - 1) // L

            def p3(b, cnt):
                ids = cd[pl.ds(b * L, L)]
                o = cnt
                for j in range(L):
                    cidj = ids[j]
                    g16 = cidj * L + iota16
                    v = plsc.load_gather(rb, [g16])
                    m = jnp.logical_and(v >= t, b * L + j < ccnt)
                    n = plsc.all_reduce_population_count(m)[0]
                    plsc.store_compressed(cv.at[pl.ds(o, L)], v,
                                          mask=m)
                    plsc.store_compressed(ci.at[pl.ds(o, L)], g16,
                                          mask=m)
                    o = o + n
                return o

            cnt = lax.fori_loop(0, nb, p3, jnp.int32(0))
            si[half] = cnt
            cv[pl.ds(cnt, L)] = neg16

    def joint_select(r0):
        cnt_a = si[0]
        cnt_b = si[1]
        nva = (cnt_a + L - 1) // L
        nvb = (cnt_b + L - 1) // L
        nvm = jnp.maximum(nva, nvb)

        def sel(k, carry):
            (av0a, av1a, ai0a, ai1a,
             av0b, av1b, ai0b, ai1b) = carry

            def scan(j, c2):
                bva, bpa, bvb, bpb = c2
                pa = j * L + iota16
                va = cv0[pl.ds(j * L, L)]
                vb = cv1[pl.ds(j * L, L)]
                beta = jnp.logical_and(va > bva, j < nva)
                betb = jnp.logical_and(vb > bvb, j < nvb)
                bva = jnp.where(beta, va, bva)
                bpa = jnp.where(beta, pa, bpa)
                bvb = jnp.where(betb, vb, bvb)
                bpb = jnp.where(betb, pa, bpb)
                return bva, bpa, bvb, bpb

            bva, bpa, bvb, bpb = lax.fori_loop(
                0, nvm, scan, (neg16, big16, neg16, big16))
            vma = jnp.max(bva)
            vmb = jnp.max(bvb)
            posa = jnp.minimum(
                jnp.min(jnp.where(bva == vma, bpa, big16)), CAND - 1)
            posb = jnp.minimum(
                jnp.min(jnp.where(bvb == vmb, bpb, big16)), CAND - 1)
            pa16 = jnp.full((L,), posa, jnp.int32)
            pb16 = jnp.full((L,), posb, jnp.int32)
            idxa = plsc.load_gather(ci0, [pa16])
            idxb = plsc.load_gather(ci1, [pb16])
            plsc.store_scatter(cv0, [pa16], neg16, mask=lane0)
            plsc.store_scatter(cv1, [pb16], neg16, mask=lane0)
            mk0 = iota16 == k
            mk1 = iota16 == k - L
            av0a = jnp.where(mk0, vma, av0a)
            av1a = jnp.where(mk1, vma, av1a)
            ai0a = jnp.where(mk0, idxa, ai0a)
            ai1a = jnp.where(mk1, idxa, ai1a)
            av0b = jnp.where(mk0, vmb, av0b)
            av1b = jnp.where(mk1, vmb, av1b)
            ai0b = jnp.where(mk0, idxb, ai0b)
            ai1b = jnp.where(mk1, idxb, ai1b)
            return (av0a, av1a, ai0a, ai1a,
                    av0b, av1b, ai0b, ai1b)

        (av0a, av1a, ai0a, ai1a,
         av0b, av1b, ai0b, ai1b) = lax.fori_loop(
            0, K, sel,
            (neg16, neg16, big16, big16,
             neg16, neg16, big16, big16))

        @pl.when(valid(r0))
        def _():
            ovals[pl.ds(r0 * KP, L)] = av0a
            ovals[pl.ds(r0 * KP + L, L)] = av1a
            oidx[pl.ds(r0 * KP, L)] = ai0a
            oidx[pl.ds(r0 * KP + L, L)] = ai1a

        @pl.when(valid(r0 + 1))
        def _():
            ovals[pl.ds((r0 + 1) * KP, L)] = av0b
            ovals[pl.ds((r0 + 1) * KP + L, L)] = av1b
            oidx[pl.ds((r0 + 1) * KP, L)] = ai0b
            oidx[pl.ds((r0 + 1) * KP + L, L)] = ai1b

    for s in range(4):
        start(s, s)

    def outer(i, _):
        r0 = i * 4
        stream_pass(r0, 0, 0)
        stream_pass(r0 + 1, 1, 1)
        start(r0 + 4, 0)
        start(r0 + 5, 1)
        joint_select(r0)
        stream_pass(r0 + 2, 2, 0)
        stream_pass(r0 + 3, 3, 1)
        start(r0 + 6, 2)
        start(r0 + 7, 3)
        joint_select(r0 + 2)
        return 0

    lax.fori_loop(0, (ROWS_PER_W + 3) // 4, outer, 0)
    pltpu.sync_copy(ovals, outv_hbm.at[pl.ds(base * KP, ROWS_PER_W * KP)])
    pltpu.sync_copy(oidx, outi_hbm.at[pl.ds(base * KP, ROWS_PER_W * KP)])


def _build_topk_sc():
    return functools.partial(
        pl.kernel,
        out_type=[
            jax.ShapeDtypeStruct((TOTAL_ROWS * KP,), jnp.float32),
            jax.ShapeDtypeStruct((TOTAL_ROWS * KP,), jnp.int32),
        ],
        mesh=plsc.VectorSubcoreMesh(core_axis_name="c", subcore_axis_name="s",
                                    num_cores=NC, num_subcores=NS),
        compiler_params=pltpu.CompilerParams(needs_layout_passes=False),
        scratch_types=[
            pltpu.VMEM((NP,), jnp.float32),
            pltpu.VMEM((NP,), jnp.float32),
            pltpu.VMEM((NP,), jnp.float32),
            pltpu.VMEM((NP,), jnp.float32),
            pltpu.VMEM((MCHUNKS,), jnp.float32),
            pltpu.VMEM((MCHUNKS,), jnp.float32),
            pltpu.VMEM((MCHUNKS,), jnp.float32),
            pltpu.VMEM((MCHUNKS,), jnp.float32),
            pltpu.VMEM((CIDCAP,), jnp.int32),
            pltpu.VMEM((CIDCAP,), jnp.int32),
            pltpu.VMEM((CAND,), jnp.float32),
            pltpu.VMEM((CAND,), jnp.float32),
            pltpu.VMEM((CAND,), jnp.int32),
            pltpu.VMEM((CAND,), jnp.int32),
            pltpu.VMEM((ROWS_PER_W * KP,), jnp.float32),
            pltpu.VMEM((ROWS_PER_W * KP,), jnp.int32),
            pltpu.SMEM((2,), jnp.int32),
            pltpu.SemaphoreType.DMA,
            pltpu.SemaphoreType.DMA,
            pltpu.SemaphoreType.DMA,
            pltpu.SemaphoreType.DMA,
            pltpu.SemaphoreType.DMA,
            pltpu.SemaphoreType.DMA,
            pltpu.SemaphoreType.DMA,
            pltpu.SemaphoreType.DMA,
        ],
    )(_topk_sc_body)


def _matmul_half(norm, a_half):
    return pl.pallas_call(
        _matmul_body,
        grid=(NH // BR, NP // BC),
        in_specs=[
            pl.BlockSpec((BR, D), lambda i, j: (i, 0)),
            pl.BlockSpec((BC, D), lambda i, j: (j, 0)),
        ],
        out_specs=[
            pl.BlockSpec((BR, BC), lambda i, j: (i, j)),
            pl.BlockSpec((BR, BC // L), lambda i, j: (i, j)),
        ],
        out_shape=[
            jax.ShapeDtypeStruct((NH, NP), jnp.float32),
            jax.ShapeDtypeStruct((NH, MCHUNKS), jnp.float32),
        ],
    )(a_half, norm)


def kernel(embeddings):
    emb_pad = jnp.pad(embeddings, ((0, NP - N), (0, 0)))
    norm = pl.pallas_call(
        _norm_body,
        out_shape=jax.ShapeDtypeStruct((NP, D), jnp.float32),
    )(emb_pad)

    topk = _build_topk_sc()
    vs, ids = [], []
    for h in range(N // NH):
        sim_h, mx_h = _matmul_half(norm, norm[h * NH:(h + 1) * NH])
        v_h, i_h = topk(sim_h, mx_h)
        vs.append(v_h.reshape(TOTAL_ROWS, KP)[:NH, :K])
        ids.append(i_h.reshape(TOTAL_ROWS, KP)[:NH, :K])
    return jnp.concatenate(vs), jnp.concatenate(ids)
